# baseline (speedup 1.0000x reference)
import jax
import jax.numpy as jnp
import numpy as np
from jax import lax
from jax.experimental import pallas as pl
from jax.experimental.pallas import tpu as pltpu
from jax.experimental.pallas import tpu_sc as plsc

N_NODES = 50000
C = 128
G = 4
GW = C // G
R_BLK = 2000

E = 250000
CHUNK = 128
K_CHUNKS = 128
KC = 32
N_ROUNDS = K_CHUNKS // KC
E_PAD = 16 * K_CHUNKS * CHUNK
N_ACC = 50176
ROWS_SUB = N_ACC // 16
ZROWS = ROWS_SUB // 8
NP = N_ACC // 4
RP = 392
NB = 4 * RP

_PAD_SRC = np.asarray((np.arange(E_PAD - E) * 37) % N_NODES, np.int32)
_PAD_DST = np.asarray(N_NODES + np.arange(E_PAD - E) % (N_ACC - N_NODES),
                      np.int32)


def _sc_agg_body(xall, src2d, dst2d, sums_out,
                 ebuf, dbuf, rows_a, rows_b, zbuf, acc_sh,
                 sem_a, sem_b):
    c = lax.axis_index("c")
    s = lax.axis_index("s")
    row_lo = s * ROWS_SUB

    zero16 = jnp.zeros((16,), jnp.float32)
    one16 = jnp.ones((16,), jnp.float32)

    def _fill(r, carry):
        zbuf[r, pl.ds(0, 16)] = zero16
        zbuf[r, pl.ds(16, 16)] = zero16
        return carry

    lax.fori_loop(0, ZROWS, _fill, 0)

    def _zero_acc():
        for q in range(ROWS_SUB // ZROWS):
            pltpu.sync_copy(zbuf, acc_sh.at[pl.ds(row_lo + q * ZROWS, ZROWS)])

    def _dma_out(plane):
        pltpu.sync_copy(acc_sh.at[pl.ds(row_lo, ROWS_SUB)],
                        sums_out.at[c, plane, pl.ds(row_lo, ROWS_SUB)])

    for g in range(G):
        _zero_acc()
        plsc.subcore_barrier()

        def _round(r, carry):
            base = s * K_CHUNKS + r * KC
            pltpu.sync_copy(src2d.at[c, pl.ds(base, KC)], ebuf)
            pltpu.sync_copy(dst2d.at[c, pl.ds(base, KC)], dbuf)

            tg = xall.at[g, c]

            bufs = (rows_a, rows_b)
            sems = (sem_a, sem_b)
            pltpu.async_copy(tg.at[ebuf.at[0]], rows_a, sem_a)
            for j in range(KC):
                buf, sem = bufs[j % 2], sems[j % 2]
                if j + 1 < KC:
                    pltpu.async_copy(tg.at[ebuf.at[j + 1]],
                                     bufs[(j + 1) % 2], sems[(j + 1) % 2])
                pltpu.make_async_copy(tg.at[ebuf.at[j]], buf, sem).wait()
                pltpu.sync_copy(buf, acc_sh.at[dbuf.at[j]], add=True)
            return carry

        lax.fori_loop(0, N_ROUNDS, _round, 0)
        plsc.subcore_barrier()
        _dma_out(g)

    _zero_acc()
    plsc.subcore_barrier()

    def _ofill(r, carry):
        rows_a[r, pl.ds(0, 16)] = one16
        rows_a[r, pl.ds(16, 16)] = one16
        return carry

    lax.fori_loop(0, CHUNK, _ofill, 0)

    def _cround(r, carry):
        base = s * K_CHUNKS + r * KC
        pltpu.sync_copy(dst2d.at[c, pl.ds(base, KC)], dbuf)
        for j in range(KC):
            pltpu.sync_copy(rows_a, acc_sh.at[dbuf.at[j]], add=True)
        return carry

    lax.fori_loop(0, N_ROUNDS, _cround, 0)
    plsc.subcore_barrier()
    _dma_out(G)


def _sc_agg(x_user, x_item, ei_u2i, ei_i2u):
    xall = jnp.stack([x_item.reshape(N_NODES, G, GW),
                      x_user.reshape(N_NODES, G, GW)])
    xall = jnp.transpose(xall, (2, 0, 1, 3))
    pad_src = jnp.asarray(_PAD_SRC)
    pad_dst = jnp.asarray(_PAD_DST)

    def prep(ei):
        src = jnp.concatenate([ei[0].astype(jnp.int32), pad_src])
        dst = jnp.concatenate([ei[1].astype(jnp.int32), pad_dst])
        return src.reshape(-1, CHUNK), dst.reshape(-1, CHUNK)

    s0, d0 = prep(ei_i2u)
    s1, d1 = prep(ei_u2i)
    src2d = jnp.stack([s0, s1])
    dst2d = jnp.stack([d0, d1])

    run = pl.kernel(
        _sc_agg_body,
        mesh=plsc.VectorSubcoreMesh(core_axis_name="c", subcore_axis_name="s",
                                    num_cores=2, num_subcores=16),
        out_type=jax.ShapeDtypeStruct((2, G + 1, N_ACC, GW), jnp.float32),
        scratch_types=[
            pltpu.VMEM((KC, CHUNK), jnp.int32),
            pltpu.VMEM((KC, CHUNK), jnp.int32),
            pltpu.VMEM((CHUNK, GW), jnp.float32),
            pltpu.VMEM((CHUNK, GW), jnp.float32),
            pltpu.VMEM((ZROWS, GW), jnp.float32),
            pltpu.VMEM_SHARED((N_ACC, GW), jnp.float32),
            pltpu.SemaphoreType.DMA,
            pltpu.SemaphoreType.DMA,
        ],
        compiler_params=pltpu.CompilerParams(use_tc_tiling_on_sc=False),
    )
    return run(xall, src2d, dst2d)


def _dense_body(sums_ref, xu_ref, xi_ref,
                ml_i2u_ref, mr_i2u_ref, bl_i2u_ref,
                ml_u2i_ref, mr_u2i_ref, bl_u2i_ref,
                lnw_u_ref, lnb_u_ref, lnw_i_ref, lnb_i_ref,
                d_ref, b_ref, out_ref):
    def one_side(rel, x_ref, ml_ref, mr_ref, bl_ref, lnw_ref, lnb_ref):
        cntp = sums_ref[rel, G]
        rc = 1.0 / jnp.maximum(cntp, 1.0)
        pcat = jnp.concatenate(
            [sums_ref[rel, g] * rc for g in range(G)], axis=1)
        y = lax.dot_general(pcat, ml_ref[...], (((1,), (0,)), ((), ())),
                            preferred_element_type=jnp.float32)
        y = y + lax.dot_general(x_ref[...], mr_ref[...], (((1,), (0,)), ((), ())),
                                preferred_element_type=jnp.float32)
        y = y + bl_ref[0]
        mu = lax.dot_general(lax.dot_general(y, d_ref[...],
                                             (((1,), (0,)), ((), ())),
                                             preferred_element_type=jnp.float32),
                             b_ref[...], (((1,), (0,)), ((), ())),
                             preferred_element_type=jnp.float32)
        d = y - mu
        var = lax.dot_general(lax.dot_general(d * d, d_ref[...],
                                              (((1,), (0,)), ((), ())),
                                              preferred_element_type=jnp.float32),
                              b_ref[...], (((1,), (0,)), ((), ())),
                              preferred_element_type=jnp.float32)
        y = d * lax.rsqrt(var + 1e-5) * lnw_ref[0] + lnb_ref[0]
        out_ref[rel] = jnp.maximum(y, 0.0)

    one_side(0, xu_ref, ml_i2u_ref, mr_i2u_ref, bl_i2u_ref, lnw_u_ref, lnb_u_ref)
    one_side(1, xi_ref, ml_u2i_ref, mr_u2i_ref, bl_u2i_ref, lnw_i_ref, lnb_i_ref)


def _pack_weights(Wl, Wr, bl, ln_w, ln_b):
    eye4 = jnp.eye(4, dtype=jnp.float32)
    ml = jnp.concatenate(
        [jnp.kron(eye4, Wl[:, g * GW:(g + 1) * GW].T) for g in range(G)])
    mr = jnp.kron(eye4, Wr.T)
    return (ml, mr, jnp.tile(bl, 4).reshape(1, 4 * C),
            jnp.tile(ln_w, 4).reshape(1, 4 * C),
            jnp.tile(ln_b, 4).reshape(1, 4 * C))


def _dense_stage(sums, x_user, x_item,
                 Wl_i2u, Wr_i2u, bl_i2u, Wl_u2i, Wr_u2i, bl_u2i,
                 ln_w_user, ln_b_user, ln_w_item, ln_b_item):
    n_blk = NP // RP
    CP = 4 * C
    ml_u, mr_u, bl_u, lnw_u, lnb_u = _pack_weights(Wl_i2u, Wr_i2u, bl_i2u,
                                                   ln_w_user, ln_b_user)
    ml_i, mr_i, bl_i, lnw_i, lnb_i = _pack_weights(Wl_u2i, Wr_u2i, bl_u2i,
                                                   ln_w_item, ln_b_item)
    dmat = jnp.kron(jnp.eye(4, dtype=jnp.float32),
                    jnp.ones((C, 1), jnp.float32)) * (1.0 / C)
    bmat = jnp.kron(jnp.eye(4, dtype=jnp.float32),
                    jnp.ones((1, C), jnp.float32))
    full = lambda shape: pl.BlockSpec(shape, lambda i: tuple(0 for _ in shape))
    out = pl.pallas_call(
        _dense_body,
        grid=(n_blk,),
        in_specs=[
            pl.BlockSpec((2, G + 1, RP, C), lambda i: (0, 0, i, 0)),
            pl.BlockSpec((RP, CP), lambda i: (i, 0)),
            pl.BlockSpec((RP, CP), lambda i: (i, 0)),
            full((CP, CP)), full((CP, CP)), full((1, CP)),
            full((CP, CP)), full((CP, CP)), full((1, CP)),
            full((1, CP)), full((1, CP)), full((1, CP)), full((1, CP)),
            full((CP, 4)), full((4, CP)),
        ],
        out_specs=pl.BlockSpec((2, RP, CP), lambda i: (0, i, 0)),
        out_shape=jax.ShapeDtypeStruct((2, N_NODES // 4, CP), jnp.float32),
    )(sums, x_user.reshape(-1, CP), x_item.reshape(-1, CP),
      ml_u, mr_u, bl_u, ml_i, mr_i, bl_i,
      lnw_u, lnb_u, lnw_i, lnb_i, dmat, bmat)
    return out.reshape(2, N_NODES, C)


def kernel(x_user, x_item, edge_index_user_to_item, edge_index_item_rev_to_user,
           Wl_u2i, bl_u2i, Wr_u2i, Wl_i2u, bl_i2u, Wr_i2u,
           ln_w_user, ln_b_user, ln_w_item, ln_b_item):
    sums = _sc_agg(x_user, x_item, edge_index_user_to_item,
                   edge_index_item_rev_to_user)
    sums = sums.reshape(2, G + 1, NP, C)
    return _dense_stage(sums, x_user, x_item,
                        Wl_i2u, Wr_i2u, bl_i2u, Wl_u2i, Wr_u2i, bl_u2i,
                        ln_w_user, ln_b_user, ln_w_item, ln_b_item)


# speedup vs baseline: 1.4575x; 1.4575x over previous
import jax
import jax.numpy as jnp
import numpy as np
from jax import lax
from jax.experimental import pallas as pl
from jax.experimental.pallas import tpu as pltpu
from jax.experimental.pallas import tpu_sc as plsc

N_NODES = 50000
C = 128
G = 4
GW = C // G
R_BLK = 2000

E = 250000
CHUNK = 128
K_CHUNKS = 128
KC = 16
N_ROUNDS = K_CHUNKS // KC
E_PAD = 16 * K_CHUNKS * CHUNK
N_ACC = 50176
ROWS_SUB = N_ACC // 16
ZROWS = ROWS_SUB // 8
NP = N_ACC // 4
RP = 392
NB = 4 * RP

_PAD_SRC = np.asarray((np.arange(E_PAD - E) * 37) % N_NODES, np.int32)
_PAD_DST = np.asarray(N_NODES + np.arange(E_PAD - E) % (N_ACC - N_NODES),
                      np.int32)


def _sc_agg_body(xall, src2d, dst2d, sums_out,
                 ebuf, dbuf, rows_a, rows_b, rows_c, zbuf, acc_sh,
                 sem_a, sem_b, sem_c):
    c = lax.axis_index("c")
    s = lax.axis_index("s")
    row_lo = s * ROWS_SUB

    zero16 = jnp.zeros((16,), jnp.float32)
    one16 = jnp.ones((16,), jnp.float32)

    def _fill(r, carry):
        zbuf[r, pl.ds(0, 16)] = zero16
        zbuf[r, pl.ds(16, 16)] = zero16
        return carry

    lax.fori_loop(0, ZROWS, _fill, 0)

    def _zero_acc():
        for q in range(ROWS_SUB // ZROWS):
            pltpu.sync_copy(zbuf, acc_sh.at[pl.ds(row_lo + q * ZROWS, ZROWS)])

    def _dma_out(plane):
        pltpu.sync_copy(acc_sh.at[pl.ds(row_lo, ROWS_SUB)],
                        sums_out.at[c, plane, pl.ds(row_lo, ROWS_SUB)])

    for g in range(G):
        _zero_acc()
        plsc.subcore_barrier()

        def _round(r, carry):
            base = s * K_CHUNKS + r * KC
            pltpu.sync_copy(src2d.at[c, pl.ds(base, KC)], ebuf)
            pltpu.sync_copy(dst2d.at[c, pl.ds(base, KC)], dbuf)

            gbase = g + c * (4 * N_NODES)

            def _ixf(j, carry2):
                for k in range(CHUNK // 16):
                    v = ebuf[j, pl.ds(k * 16, 16)]
                    ebuf[j, pl.ds(k * 16, 16)] = v * 4 + gbase
                return carry2

            lax.fori_loop(0, KC, _ixf, 0)

            bufs = (rows_a, rows_b, rows_c)
            sems = (sem_a, sem_b, sem_c)
            pltpu.async_copy(xall.at[ebuf.at[0]], rows_a, sem_a)
            pltpu.async_copy(xall.at[ebuf.at[1]], rows_b, sem_b)
            for j in range(KC):
                buf, sem = bufs[j % 3], sems[j % 3]
                if j + 2 < KC:
                    pltpu.async_copy(xall.at[ebuf.at[j + 2]],
                                     bufs[(j + 2) % 3], sems[(j + 2) % 3])
                pltpu.make_async_copy(xall.at[ebuf.at[j]], buf, sem).wait()
                pltpu.sync_copy(buf, acc_sh.at[dbuf.at[j]], add=True)
            return carry

        lax.fori_loop(0, N_ROUNDS, _round, 0)
        plsc.subcore_barrier()
        _dma_out(g)

    _zero_acc()
    plsc.subcore_barrier()

    def _ofill(r, carry):
        rows_a[r, pl.ds(0, 16)] = one16
        rows_a[r, pl.ds(16, 16)] = one16
        return carry

    lax.fori_loop(0, CHUNK, _ofill, 0)

    def _cround(r, carry):
        base = s * K_CHUNKS + r * KC
        pltpu.sync_copy(dst2d.at[c, pl.ds(base, KC)], dbuf)
        for j in range(KC):
            pltpu.sync_copy(rows_a, acc_sh.at[dbuf.at[j]], add=True)
        return carry

    lax.fori_loop(0, N_ROUNDS, _cround, 0)
    plsc.subcore_barrier()
    _dma_out(G)


def _sc_agg(x_user, x_item, ei_u2i, ei_i2u):
    xall = jnp.concatenate([x_item, x_user]).reshape(-1, GW)
    pad_src = jnp.asarray(_PAD_SRC)
    pad_dst = jnp.asarray(_PAD_DST)

    def prep(ei):
        src = jnp.concatenate([ei[0].astype(jnp.int32), pad_src])
        dst = jnp.concatenate([ei[1].astype(jnp.int32), pad_dst])
        return src.reshape(-1, CHUNK), dst.reshape(-1, CHUNK)

    s0, d0 = prep(ei_i2u)
    s1, d1 = prep(ei_u2i)
    src2d = jnp.stack([s0, s1])
    dst2d = jnp.stack([d0, d1])

    run = pl.kernel(
        _sc_agg_body,
        mesh=plsc.VectorSubcoreMesh(core_axis_name="c", subcore_axis_name="s",
                                    num_cores=2, num_subcores=16),
        out_type=jax.ShapeDtypeStruct((2, G + 1, N_ACC, GW), jnp.float32),
        scratch_types=[
            pltpu.VMEM((KC, CHUNK), jnp.int32),
            pltpu.VMEM((KC, CHUNK), jnp.int32),
            pltpu.VMEM((CHUNK, GW), jnp.float32),
            pltpu.VMEM((CHUNK, GW), jnp.float32),
            pltpu.VMEM((CHUNK, GW), jnp.float32),
            pltpu.VMEM((ZROWS, GW), jnp.float32),
            pltpu.VMEM_SHARED((N_ACC, GW), jnp.float32),
            pltpu.SemaphoreType.DMA,
            pltpu.SemaphoreType.DMA,
            pltpu.SemaphoreType.DMA,
        ],
        compiler_params=pltpu.CompilerParams(use_tc_tiling_on_sc=False),
    )
    return run(xall, src2d, dst2d)


def _dense_body(sums_ref, xu_ref, xi_ref,
                ml_i2u_ref, mr_i2u_ref, bl_i2u_ref,
                ml_u2i_ref, mr_u2i_ref, bl_u2i_ref,
                lnw_u_ref, lnb_u_ref, lnw_i_ref, lnb_i_ref,
                d_ref, b_ref, out_ref):
    def one_side(rel, x_ref, ml_ref, mr_ref, bl_ref, lnw_ref, lnb_ref):
        cntp = sums_ref[rel, G]
        rc = 1.0 / jnp.maximum(cntp, 1.0)
        pcat = jnp.concatenate(
            [sums_ref[rel, g] * rc for g in range(G)], axis=1)
        y = lax.dot_general(pcat, ml_ref[...], (((1,), (0,)), ((), ())),
                            preferred_element_type=jnp.float32)
        y = y + lax.dot_general(x_ref[...], mr_ref[...], (((1,), (0,)), ((), ())),
                                preferred_element_type=jnp.float32)
        y = y + bl_ref[0]
        mu = lax.dot_general(lax.dot_general(y, d_ref[...],
                                             (((1,), (0,)), ((), ())),
                                             preferred_element_type=jnp.float32),
                             b_ref[...], (((1,), (0,)), ((), ())),
                             preferred_element_type=jnp.float32)
        d = y - mu
        var = lax.dot_general(lax.dot_general(d * d, d_ref[...],
                                              (((1,), (0,)), ((), ())),
                                              preferred_element_type=jnp.float32),
                              b_ref[...], (((1,), (0,)), ((), ())),
                              preferred_element_type=jnp.float32)
        y = d * lax.rsqrt(var + 1e-5) * lnw_ref[0] + lnb_ref[0]
        out_ref[rel] = jnp.maximum(y, 0.0)

    one_side(0, xu_ref, ml_i2u_ref, mr_i2u_ref, bl_i2u_ref, lnw_u_ref, lnb_u_ref)
    one_side(1, xi_ref, ml_u2i_ref, mr_u2i_ref, bl_u2i_ref, lnw_i_ref, lnb_i_ref)


def _pack_weights(Wl, Wr, bl, ln_w, ln_b):
    eye4 = jnp.eye(4, dtype=jnp.float32)
    ml = jnp.concatenate(
        [jnp.kron(eye4, Wl[:, g * GW:(g + 1) * GW].T) for g in range(G)])
    mr = jnp.kron(eye4, Wr.T)
    return (ml, mr, jnp.tile(bl, 4).reshape(1, 4 * C),
            jnp.tile(ln_w, 4).reshape(1, 4 * C),
            jnp.tile(ln_b, 4).reshape(1, 4 * C))


def _dense_stage(sums, x_user, x_item,
                 Wl_i2u, Wr_i2u, bl_i2u, Wl_u2i, Wr_u2i, bl_u2i,
                 ln_w_user, ln_b_user, ln_w_item, ln_b_item):
    n_blk = NP // RP
    CP = 4 * C
    ml_u, mr_u, bl_u, lnw_u, lnb_u = _pack_weights(Wl_i2u, Wr_i2u, bl_i2u,
                                                   ln_w_user, ln_b_user)
    ml_i, mr_i, bl_i, lnw_i, lnb_i = _pack_weights(Wl_u2i, Wr_u2i, bl_u2i,
                                                   ln_w_item, ln_b_item)
    dmat = jnp.kron(jnp.eye(4, dtype=jnp.float32),
                    jnp.ones((C, 1), jnp.float32)) * (1.0 / C)
    bmat = jnp.kron(jnp.eye(4, dtype=jnp.float32),
                    jnp.ones((1, C), jnp.float32))
    full = lambda shape: pl.BlockSpec(shape, lambda i: tuple(0 for _ in shape))
    out = pl.pallas_call(
        _dense_body,
        grid=(n_blk,),
        in_specs=[
            pl.BlockSpec((2, G + 1, RP, C), lambda i: (0, 0, i, 0)),
            pl.BlockSpec((RP, CP), lambda i: (i, 0)),
            pl.BlockSpec((RP, CP), lambda i: (i, 0)),
            full((CP, CP)), full((CP, CP)), full((1, CP)),
            full((CP, CP)), full((CP, CP)), full((1, CP)),
            full((1, CP)), full((1, CP)), full((1, CP)), full((1, CP)),
            full((CP, 4)), full((4, CP)),
        ],
        out_specs=pl.BlockSpec((2, RP, CP), lambda i: (0, i, 0)),
        out_shape=jax.ShapeDtypeStruct((2, N_NODES // 4, CP), jnp.float32),
    )(sums, x_user.reshape(-1, CP), x_item.reshape(-1, CP),
      ml_u, mr_u, bl_u, ml_i, mr_i, bl_i,
      lnw_u, lnb_u, lnw_i, lnb_i, dmat, bmat)
    return out.reshape(2, N_NODES, C)


def kernel(x_user, x_item, edge_index_user_to_item, edge_index_item_rev_to_user,
           Wl_u2i, bl_u2i, Wr_u2i, Wl_i2u, bl_i2u, Wr_i2u,
           ln_w_user, ln_b_user, ln_w_item, ln_b_item):
    sums = _sc_agg(x_user, x_item, edge_index_user_to_item,
                   edge_index_item_rev_to_user)
    sums = sums.reshape(2, G + 1, NP, C)
    return _dense_stage(sums, x_user, x_item,
                        Wl_i2u, Wr_i2u, bl_i2u, Wl_u2i, Wr_u2i, bl_u2i,
                        ln_w_user, ln_b_user, ln_w_item, ln_b_item)


# speedup vs baseline: 1.5779x; 1.0826x over previous
import jax
import jax.numpy as jnp
import numpy as np
from jax import lax
from jax.experimental import pallas as pl
from jax.experimental.pallas import tpu as pltpu
from jax.experimental.pallas import tpu_sc as plsc

N_NODES = 50000
C = 128
G = 4
GW = C // G
R_BLK = 2000

E = 250000
CHUNK = 128
K_CHUNKS = 128
KC = 16
N_ROUNDS = K_CHUNKS // KC
E_PAD = 16 * K_CHUNKS * CHUNK
N_ACC = 50176
ROWS_SUB = N_ACC // 16
ZROWS = ROWS_SUB // 28
NP = N_ACC // 4
RP = 392
NB = 4 * RP

_PAD_SRC = np.asarray((np.arange(E_PAD - E) * 37) % N_NODES, np.int32)
_PAD_DST = np.asarray(N_NODES + np.arange(E_PAD - E) % (N_ACC - N_NODES),
                      np.int32)


def _sc_agg_body(xall, ecomb, sums_out,
                 ebuf, dbuf, rows_a, rows_b, rows_c, rows_d, zbuf, acc_sh,
                 sem_a, sem_b, sem_c, sem_d):
    c = lax.axis_index("c")
    s = lax.axis_index("s")
    row_lo = s * ROWS_SUB

    zero16 = jnp.zeros((16,), jnp.float32)
    one16 = jnp.ones((16,), jnp.float32)

    def _fill(r, carry):
        zbuf[r, pl.ds(0, 16)] = zero16
        zbuf[r, pl.ds(16, 16)] = zero16
        return carry

    lax.fori_loop(0, ZROWS, _fill, 0)

    def _zero_acc():
        for q in range(ROWS_SUB // ZROWS):
            pltpu.sync_copy(zbuf, acc_sh.at[pl.ds(row_lo + q * ZROWS, ZROWS)])

    def _dma_out(plane):
        pltpu.sync_copy(acc_sh.at[pl.ds(row_lo, ROWS_SUB)],
                        sums_out.at[c, plane, pl.ds(row_lo, ROWS_SUB)])

    for g in range(G):
        _zero_acc()
        plsc.subcore_barrier()

        def _round(r, carry):
            base = s * K_CHUNKS + r * KC
            pltpu.sync_copy(ecomb.at[c, 0, pl.ds(base, KC)], ebuf)
            pltpu.sync_copy(ecomb.at[c, 1, pl.ds(base, KC)], dbuf)

            gbase = g + c * (4 * N_NODES)

            def _ixf(j, carry2):
                for k in range(CHUNK // 16):
                    v = ebuf[j, pl.ds(k * 16, 16)]
                    ebuf[j, pl.ds(k * 16, 16)] = v * 4 + gbase
                return carry2

            lax.fori_loop(0, KC, _ixf, 0)

            bufs = (rows_a, rows_b, rows_c, rows_d)
            sems = (sem_a, sem_b, sem_c, sem_d)
            for q in range(3):
                pltpu.async_copy(xall.at[ebuf.at[q]], bufs[q], sems[q])
            for j in range(KC):
                buf, sem = bufs[j % 4], sems[j % 4]
                if j + 3 < KC:
                    pltpu.async_copy(xall.at[ebuf.at[j + 3]],
                                     bufs[(j + 3) % 4], sems[(j + 3) % 4])
                pltpu.make_async_copy(xall.at[ebuf.at[j]], buf, sem).wait()
                pltpu.sync_copy(buf, acc_sh.at[dbuf.at[j]], add=True)
            return carry

        lax.fori_loop(0, N_ROUNDS, _round, 0)
        plsc.subcore_barrier()
        _dma_out(g)

    _zero_acc()
    plsc.subcore_barrier()

    def _ofill(r, carry):
        rows_a[r, pl.ds(0, 16)] = one16
        rows_a[r, pl.ds(16, 16)] = one16
        return carry

    lax.fori_loop(0, CHUNK, _ofill, 0)

    def _cround(r, carry):
        base = s * K_CHUNKS + r * KC
        pltpu.sync_copy(ecomb.at[c, 1, pl.ds(base, KC)], dbuf)
        for j in range(KC):
            pltpu.sync_copy(rows_a, acc_sh.at[dbuf.at[j]], add=True)
        return carry

    lax.fori_loop(0, N_ROUNDS, _cround, 0)
    plsc.subcore_barrier()
    _dma_out(G)


def _sc_agg(x_user, x_item, ei_u2i, ei_i2u):
    xall = jnp.concatenate([x_item, x_user]).reshape(-1, GW)
    pad_blk = jnp.asarray(np.broadcast_to(np.stack([_PAD_SRC, _PAD_DST]),
                                          (2, 2, E_PAD - E)))
    stacked = jnp.stack([ei_i2u, ei_u2i]).astype(jnp.int32)
    ecomb = jnp.concatenate([stacked, pad_blk], axis=2)
    ecomb = ecomb.reshape(2, 2, -1, CHUNK)

    run = pl.kernel(
        _sc_agg_body,
        mesh=plsc.VectorSubcoreMesh(core_axis_name="c", subcore_axis_name="s",
                                    num_cores=2, num_subcores=16),
        out_type=jax.ShapeDtypeStruct((2, G + 1, N_ACC, GW), jnp.float32),
        scratch_types=[
            pltpu.VMEM((KC, CHUNK), jnp.int32),
            pltpu.VMEM((KC, CHUNK), jnp.int32),
            pltpu.VMEM((CHUNK, GW), jnp.float32),
            pltpu.VMEM((CHUNK, GW), jnp.float32),
            pltpu.VMEM((CHUNK, GW), jnp.float32),
            pltpu.VMEM((CHUNK, GW), jnp.float32),
            pltpu.VMEM((ZROWS, GW), jnp.float32),
            pltpu.VMEM_SHARED((N_ACC, GW), jnp.float32),
            pltpu.SemaphoreType.DMA,
            pltpu.SemaphoreType.DMA,
            pltpu.SemaphoreType.DMA,
            pltpu.SemaphoreType.DMA,
        ],
        compiler_params=pltpu.CompilerParams(use_tc_tiling_on_sc=False),
    )
    return run(xall, ecomb)


def _dense_body(sums_ref, xu_ref, xi_ref,
                ml_i2u_ref, mr_i2u_ref, bl_i2u_ref,
                ml_u2i_ref, mr_u2i_ref, bl_u2i_ref,
                lnw_u_ref, lnb_u_ref, lnw_i_ref, lnb_i_ref,
                d_ref, b_ref, out_ref):
    def one_side(rel, x_ref, ml_ref, mr_ref, bl_ref, lnw_ref, lnb_ref):
        cntp = sums_ref[rel, G]
        rc = 1.0 / jnp.maximum(cntp, 1.0)
        pcat = jnp.concatenate(
            [sums_ref[rel, g] * rc for g in range(G)], axis=1)
        y = lax.dot_general(pcat, ml_ref[...], (((1,), (0,)), ((), ())),
                            preferred_element_type=jnp.float32)
        y = y + lax.dot_general(x_ref[...], mr_ref[...], (((1,), (0,)), ((), ())),
                                preferred_element_type=jnp.float32)
        y = y + bl_ref[0]
        mu = lax.dot_general(lax.dot_general(y, d_ref[...],
                                             (((1,), (0,)), ((), ())),
                                             preferred_element_type=jnp.float32),
                             b_ref[...], (((1,), (0,)), ((), ())),
                             preferred_element_type=jnp.float32)
        d = y - mu
        var = lax.dot_general(lax.dot_general(d * d, d_ref[...],
                                              (((1,), (0,)), ((), ())),
                                              preferred_element_type=jnp.float32),
                              b_ref[...], (((1,), (0,)), ((), ())),
                              preferred_element_type=jnp.float32)
        y = d * lax.rsqrt(var + 1e-5) * lnw_ref[0] + lnb_ref[0]
        out_ref[rel] = jnp.maximum(y, 0.0)

    one_side(0, xu_ref, ml_i2u_ref, mr_i2u_ref, bl_i2u_ref, lnw_u_ref, lnb_u_ref)
    one_side(1, xi_ref, ml_u2i_ref, mr_u2i_ref, bl_u2i_ref, lnw_i_ref, lnb_i_ref)


def _pack_weights(Wl, Wr, bl, ln_w, ln_b):
    eye4 = jnp.eye(4, dtype=jnp.float32)
    ml = jnp.concatenate(
        [jnp.kron(eye4, Wl[:, g * GW:(g + 1) * GW].T) for g in range(G)])
    mr = jnp.kron(eye4, Wr.T)
    return (ml, mr, jnp.tile(bl, 4).reshape(1, 4 * C),
            jnp.tile(ln_w, 4).reshape(1, 4 * C),
            jnp.tile(ln_b, 4).reshape(1, 4 * C))


def _dense_stage(sums, x_user, x_item,
                 Wl_i2u, Wr_i2u, bl_i2u, Wl_u2i, Wr_u2i, bl_u2i,
                 ln_w_user, ln_b_user, ln_w_item, ln_b_item):
    n_blk = NP // RP
    CP = 4 * C
    ml_u, mr_u, bl_u, lnw_u, lnb_u = _pack_weights(Wl_i2u, Wr_i2u, bl_i2u,
                                                   ln_w_user, ln_b_user)
    ml_i, mr_i, bl_i, lnw_i, lnb_i = _pack_weights(Wl_u2i, Wr_u2i, bl_u2i,
                                                   ln_w_item, ln_b_item)
    dmat = jnp.kron(jnp.eye(4, dtype=jnp.float32),
                    jnp.ones((C, 1), jnp.float32)) * (1.0 / C)
    bmat = jnp.kron(jnp.eye(4, dtype=jnp.float32),
                    jnp.ones((1, C), jnp.float32))
    full = lambda shape: pl.BlockSpec(shape, lambda i: tuple(0 for _ in shape))
    out = pl.pallas_call(
        _dense_body,
        grid=(n_blk,),
        in_specs=[
            pl.BlockSpec((2, G + 1, RP, C), lambda i: (0, 0, i, 0)),
            pl.BlockSpec((RP, CP), lambda i: (i, 0)),
            pl.BlockSpec((RP, CP), lambda i: (i, 0)),
            full((CP, CP)), full((CP, CP)), full((1, CP)),
            full((CP, CP)), full((CP, CP)), full((1, CP)),
            full((1, CP)), full((1, CP)), full((1, CP)), full((1, CP)),
            full((CP, 4)), full((4, CP)),
        ],
        out_specs=pl.BlockSpec((2, RP, CP), lambda i: (0, i, 0)),
        out_shape=jax.ShapeDtypeStruct((2, N_NODES // 4, CP), jnp.float32),
    )(sums, x_user.reshape(-1, CP), x_item.reshape(-1, CP),
      ml_u, mr_u, bl_u, ml_i, mr_i, bl_i,
      lnw_u, lnb_u, lnw_i, lnb_i, dmat, bmat)
    return out.reshape(2, N_NODES, C)


def kernel(x_user, x_item, edge_index_user_to_item, edge_index_item_rev_to_user,
           Wl_u2i, bl_u2i, Wr_u2i, Wl_i2u, bl_i2u, Wr_i2u,
           ln_w_user, ln_b_user, ln_w_item, ln_b_item):
    sums = _sc_agg(x_user, x_item, edge_index_user_to_item,
                   edge_index_item_rev_to_user)
    sums = sums.reshape(2, G + 1, NP, C)
    return _dense_stage(sums, x_user, x_item,
                        Wl_i2u, Wr_i2u, bl_i2u, Wl_u2i, Wr_u2i, bl_u2i,
                        ln_w_user, ln_b_user, ln_w_item, ln_b_item)


# speedup vs baseline: 1.6054x; 1.0175x over previous
import jax
import jax.numpy as jnp
import numpy as np
from jax import lax
from jax.experimental import pallas as pl
from jax.experimental.pallas import tpu as pltpu
from jax.experimental.pallas import tpu_sc as plsc

N_NODES = 50000
C = 128
G = 4
GW = C // G
R_BLK = 2000

E = 250000
CHUNK = 128
K_CHUNKS = 128
KC = 16
N_ROUNDS = K_CHUNKS // KC
E_PAD = 16 * K_CHUNKS * CHUNK
N_ACC = 50176
ROWS_SUB = N_ACC // 16
ZROWS = ROWS_SUB // 28
NP = N_ACC // 4
RP = 392
NB = 4 * RP

_PAD_SRC = np.asarray((np.arange(E_PAD - E) * 37) % N_NODES, np.int32)
_PAD_DST = np.asarray(N_NODES + np.arange(E_PAD - E) % (N_ACC - N_NODES),
                      np.int32)


def _sc_agg_body(xall, ecomb, sums_out,
                 ebuf, dbuf, rows_a, rows_b, rows_c, rows_d, rows_e, zbuf,
                 acc_sh, sem_a, sem_b, sem_c, sem_d, sem_e):
    c = lax.axis_index("c")
    s = lax.axis_index("s")
    row_lo = s * ROWS_SUB

    zero16 = jnp.zeros((16,), jnp.float32)
    one16 = jnp.ones((16,), jnp.float32)

    def _fill(r, carry):
        zbuf[r, pl.ds(0, 16)] = zero16
        zbuf[r, pl.ds(16, 16)] = zero16
        return carry

    lax.fori_loop(0, ZROWS, _fill, 0)

    def _zero_acc():
        for q in range(ROWS_SUB // ZROWS):
            pltpu.sync_copy(zbuf, acc_sh.at[pl.ds(row_lo + q * ZROWS, ZROWS)])

    def _dma_out(plane):
        pltpu.sync_copy(acc_sh.at[pl.ds(row_lo, ROWS_SUB)],
                        sums_out.at[c, plane, pl.ds(row_lo, ROWS_SUB)])

    for g in range(G):
        _zero_acc()
        plsc.subcore_barrier()

        def _round(r, carry):
            base = s * K_CHUNKS + r * KC
            pltpu.sync_copy(ecomb.at[c, 0, pl.ds(base, KC)], ebuf)
            pltpu.sync_copy(ecomb.at[c, 1, pl.ds(base, KC)], dbuf)

            gbase = g + c * (4 * N_NODES)

            def _ixf(j, carry2):
                for k in range(CHUNK // 16):
                    v = ebuf[j, pl.ds(k * 16, 16)]
                    ebuf[j, pl.ds(k * 16, 16)] = v * 4 + gbase
                return carry2

            lax.fori_loop(0, KC, _ixf, 0)

            bufs = (rows_a, rows_b, rows_c, rows_d, rows_e)
            sems = (sem_a, sem_b, sem_c, sem_d, sem_e)
            for q in range(4):
                pltpu.async_copy(xall.at[ebuf.at[q]], bufs[q], sems[q])
            for j in range(KC):
                buf, sem = bufs[j % 5], sems[j % 5]
                if j + 4 < KC:
                    pltpu.async_copy(xall.at[ebuf.at[j + 4]],
                                     bufs[(j + 4) % 5], sems[(j + 4) % 5])
                pltpu.make_async_copy(xall.at[ebuf.at[j]], buf, sem).wait()
                pltpu.sync_copy(buf, acc_sh.at[dbuf.at[j]], add=True)
            return carry

        lax.fori_loop(0, N_ROUNDS, _round, 0)
        plsc.subcore_barrier()
        _dma_out(g)

    _zero_acc()
    plsc.subcore_barrier()

    def _ofill(r, carry):
        rows_a[r, pl.ds(0, 16)] = one16
        rows_a[r, pl.ds(16, 16)] = one16
        return carry

    lax.fori_loop(0, CHUNK, _ofill, 0)

    def _cround(r, carry):
        base = s * K_CHUNKS + r * KC
        pltpu.sync_copy(ecomb.at[c, 1, pl.ds(base, KC)], dbuf)
        for j in range(KC):
            pltpu.sync_copy(rows_a, acc_sh.at[dbuf.at[j]], add=True)
        return carry

    lax.fori_loop(0, N_ROUNDS, _cround, 0)
    plsc.subcore_barrier()
    _dma_out(G)


def _sc_agg(x_user, x_item, ei_u2i, ei_i2u):
    xall = jnp.concatenate([x_item, x_user]).reshape(-1, GW)
    pad_blk = jnp.asarray(np.broadcast_to(np.stack([_PAD_SRC, _PAD_DST]),
                                          (2, 2, E_PAD - E)))
    stacked = jnp.stack([ei_i2u, ei_u2i]).astype(jnp.int32)
    ecomb = jnp.concatenate([stacked, pad_blk], axis=2)
    ecomb = ecomb.reshape(2, 2, -1, CHUNK)

    run = pl.kernel(
        _sc_agg_body,
        mesh=plsc.VectorSubcoreMesh(core_axis_name="c", subcore_axis_name="s",
                                    num_cores=2, num_subcores=16),
        out_type=jax.ShapeDtypeStruct((2, G + 1, N_ACC, GW), jnp.float32),
        scratch_types=[
            pltpu.VMEM((KC, CHUNK), jnp.int32),
            pltpu.VMEM((KC, CHUNK), jnp.int32),
            pltpu.VMEM((CHUNK, GW), jnp.float32),
            pltpu.VMEM((CHUNK, GW), jnp.float32),
            pltpu.VMEM((CHUNK, GW), jnp.float32),
            pltpu.VMEM((CHUNK, GW), jnp.float32),
            pltpu.VMEM((CHUNK, GW), jnp.float32),
            pltpu.VMEM((ZROWS, GW), jnp.float32),
            pltpu.VMEM_SHARED((N_ACC, GW), jnp.float32),
            pltpu.SemaphoreType.DMA,
            pltpu.SemaphoreType.DMA,
            pltpu.SemaphoreType.DMA,
            pltpu.SemaphoreType.DMA,
            pltpu.SemaphoreType.DMA,
        ],
        compiler_params=pltpu.CompilerParams(use_tc_tiling_on_sc=False),
    )
    return run(xall, ecomb)


def _dense_body(sums_ref, xu_ref, xi_ref,
                ml_i2u_ref, mr_i2u_ref, bl_i2u_ref,
                ml_u2i_ref, mr_u2i_ref, bl_u2i_ref,
                lnw_u_ref, lnb_u_ref, lnw_i_ref, lnb_i_ref,
                d_ref, b_ref, out_ref):
    def one_side(rel, x_ref, ml_ref, mr_ref, bl_ref, lnw_ref, lnb_ref):
        cntp = sums_ref[rel, G]
        rc = 1.0 / jnp.maximum(cntp, 1.0)
        pcat = jnp.concatenate(
            [sums_ref[rel, g] * rc for g in range(G)], axis=1)
        y = lax.dot_general(pcat, ml_ref[...], (((1,), (0,)), ((), ())),
                            preferred_element_type=jnp.float32)
        y = y + lax.dot_general(x_ref[...], mr_ref[...], (((1,), (0,)), ((), ())),
                                preferred_element_type=jnp.float32)
        y = y + bl_ref[0]
        mu = lax.dot_general(lax.dot_general(y, d_ref[...],
                                             (((1,), (0,)), ((), ())),
                                             preferred_element_type=jnp.float32),
                             b_ref[...], (((1,), (0,)), ((), ())),
                             preferred_element_type=jnp.float32)
        d = y - mu
        var = lax.dot_general(lax.dot_general(d * d, d_ref[...],
                                              (((1,), (0,)), ((), ())),
                                              preferred_element_type=jnp.float32),
                              b_ref[...], (((1,), (0,)), ((), ())),
                              preferred_element_type=jnp.float32)
        y = d * lax.rsqrt(var + 1e-5) * lnw_ref[0] + lnb_ref[0]
        out_ref[rel] = jnp.maximum(y, 0.0)

    one_side(0, xu_ref, ml_i2u_ref, mr_i2u_ref, bl_i2u_ref, lnw_u_ref, lnb_u_ref)
    one_side(1, xi_ref, ml_u2i_ref, mr_u2i_ref, bl_u2i_ref, lnw_i_ref, lnb_i_ref)


def _pack_weights(Wl, Wr, bl, ln_w, ln_b):
    eye4 = jnp.eye(4, dtype=jnp.float32)
    ml = jnp.concatenate(
        [jnp.kron(eye4, Wl[:, g * GW:(g + 1) * GW].T) for g in range(G)])
    mr = jnp.kron(eye4, Wr.T)
    return (ml, mr, jnp.tile(bl, 4).reshape(1, 4 * C),
            jnp.tile(ln_w, 4).reshape(1, 4 * C),
            jnp.tile(ln_b, 4).reshape(1, 4 * C))


def _dense_stage(sums, x_user, x_item,
                 Wl_i2u, Wr_i2u, bl_i2u, Wl_u2i, Wr_u2i, bl_u2i,
                 ln_w_user, ln_b_user, ln_w_item, ln_b_item):
    n_blk = NP // RP
    CP = 4 * C
    ml_u, mr_u, bl_u, lnw_u, lnb_u = _pack_weights(Wl_i2u, Wr_i2u, bl_i2u,
                                                   ln_w_user, ln_b_user)
    ml_i, mr_i, bl_i, lnw_i, lnb_i = _pack_weights(Wl_u2i, Wr_u2i, bl_u2i,
                                                   ln_w_item, ln_b_item)
    dmat = jnp.kron(jnp.eye(4, dtype=jnp.float32),
                    jnp.ones((C, 1), jnp.float32)) * (1.0 / C)
    bmat = jnp.kron(jnp.eye(4, dtype=jnp.float32),
                    jnp.ones((1, C), jnp.float32))
    full = lambda shape: pl.BlockSpec(shape, lambda i: tuple(0 for _ in shape))
    out = pl.pallas_call(
        _dense_body,
        grid=(n_blk,),
        in_specs=[
            pl.BlockSpec((2, G + 1, RP, C), lambda i: (0, 0, i, 0)),
            pl.BlockSpec((RP, CP), lambda i: (i, 0)),
            pl.BlockSpec((RP, CP), lambda i: (i, 0)),
            full((CP, CP)), full((CP, CP)), full((1, CP)),
            full((CP, CP)), full((CP, CP)), full((1, CP)),
            full((1, CP)), full((1, CP)), full((1, CP)), full((1, CP)),
            full((CP, 4)), full((4, CP)),
        ],
        out_specs=pl.BlockSpec((2, RP, CP), lambda i: (0, i, 0)),
        out_shape=jax.ShapeDtypeStruct((2, N_NODES // 4, CP), jnp.float32),
    )(sums, x_user.reshape(-1, CP), x_item.reshape(-1, CP),
      ml_u, mr_u, bl_u, ml_i, mr_i, bl_i,
      lnw_u, lnb_u, lnw_i, lnb_i, dmat, bmat)
    return out.reshape(2, N_NODES, C)


def kernel(x_user, x_item, edge_index_user_to_item, edge_index_item_rev_to_user,
           Wl_u2i, bl_u2i, Wr_u2i, Wl_i2u, bl_i2u, Wr_i2u,
           ln_w_user, ln_b_user, ln_w_item, ln_b_item):
    sums = _sc_agg(x_user, x_item, edge_index_user_to_item,
                   edge_index_item_rev_to_user)
    sums = sums.reshape(2, G + 1, NP, C)
    return _dense_stage(sums, x_user, x_item,
                        Wl_i2u, Wr_i2u, bl_i2u, Wl_u2i, Wr_u2i, bl_u2i,
                        ln_w_user, ln_b_user, ln_w_item, ln_b_item)
